# baseline (speedup 1.0000x reference)
import jax
import jax.numpy as jnp
from jax.experimental import pallas as pl
from jax.experimental.pallas import tpu as pltpu
from jax.experimental.pallas import tpu_sc as plsc

BT = 128
FCH = 1024
SC_W = 128
COL_SPLIT = 4


def _sc_mesh():
    return plsc.VectorSubcoreMesh(core_axis_name="c", subcore_axis_name="s")


def _expand4(idx):
    return (
        idx[:, None] * COL_SPLIT + jnp.arange(COL_SPLIT, dtype=jnp.int32)[None, :]
    ).reshape(1, -1)


def _sc_gather_rows(src, idx):
    n_out = idx.shape[0]
    _, d = src.shape
    ds = d // COL_SPLIT
    srcv = src.reshape(-1, ds)
    nx = n_out * COL_SPLIT
    idx4 = _expand4(idx)

    @pl.kernel(out_type=jax.ShapeDtypeStruct((nx, ds), src.dtype), mesh=_sc_mesh())
    def k(s_hbm, i_hbm, o_hbm):
        def body(i_vmem, o_vmem):
            pltpu.sync_copy(s_hbm.at[i_vmem.at[0]], o_vmem)

        pltpu.emit_pipeline(
            body,
            grid=(nx // SC_W,),
            in_specs=[pl.BlockSpec((1, SC_W), lambda i: (0, i))],
            out_specs=[pl.BlockSpec((SC_W, ds), lambda i: (i, 0))],
            core_axis_name=("c", "s"),
            dimension_semantics=(pltpu.PARALLEL,),
        )(i_hbm, o_hbm)

    return k(srcv, idx4).reshape(n_out, d)


def _sc_gather_tw(tw128, idx):
    n_out = idx.shape[0]
    lanes = tw128.shape[1]

    @pl.kernel(
        out_type=jax.ShapeDtypeStruct((n_out, lanes), tw128.dtype), mesh=_sc_mesh()
    )
    def k(s_hbm, i_hbm, o_hbm):
        def body(i_vmem, o_vmem):
            pltpu.sync_copy(s_hbm.at[i_vmem.at[0]], o_vmem)

        pltpu.emit_pipeline(
            body,
            grid=(n_out // SC_W,),
            in_specs=[pl.BlockSpec((1, SC_W), lambda i: (0, i))],
            out_specs=[pl.BlockSpec((SC_W, lanes), lambda i: (i, 0))],
            core_axis_name=("c", "s"),
            dimension_semantics=(pltpu.PARALLEL,),
        )(i_hbm, o_hbm)

    return k(tw128, idx.reshape(1, n_out))


def _fused_body(meta_ref, x_ref, tw_ref, b_ref, w1_hbm, w2_hbm, y_ref,
                w1b, w2b, w1c, w2c, sems):
    cap, d = x_ref.shape
    e_total, f, _ = w1_hbm.shape
    fch = w1b.shape[1]
    nf = f // fch
    k_total = e_total * nf

    def issue(k):
        slot = jax.lax.rem(k, 2)
        e = k // nf
        fc = jax.lax.rem(k, nf)
        pltpu.make_async_copy(
            w1_hbm.at[e, pl.ds(fc * fch, fch), :], w1b.at[slot], sems.at[slot, 0]
        ).start()
        pltpu.make_async_copy(
            w2_hbm.at[e, pl.ds(fc * fch, fch), :], w2b.at[slot], sems.at[slot, 1]
        ).start()

    y_ref[...] = jnp.zeros_like(y_ref)
    issue(0)

    def chunk_body(k, carry):
        slot = jax.lax.rem(k, 2)
        e = k // nf
        fc = jax.lax.rem(k, nf)

        @pl.when(k + 1 < k_total)
        def _():
            issue(k + 1)

        pltpu.make_async_copy(
            w1_hbm.at[e, pl.ds(fc * fch, fch), :], w1b.at[slot], sems.at[slot, 0]
        ).wait()
        pltpu.make_async_copy(
            w2_hbm.at[e, pl.ds(fc * fch, fch), :], w2b.at[slot], sems.at[slot, 1]
        ).wait()
        w1c[...] = w1b[slot].astype(jnp.bfloat16)
        w2c[...] = w2b[slot].astype(jnp.bfloat16)

        off8 = meta_ref[0, e]
        n = meta_ref[1, e]
        ntile = (n + BT - 1) // BT

        def tile_body(ti, c2):
            ob = off8 * 8 + ti * BT
            rows = ob + jax.lax.broadcasted_iota(jnp.int32, (BT, 1), 0)
            valid = rows < off8 * 8 + n
            xb = jnp.where(valid, x_ref[pl.ds(ob, BT), :], 0.0).astype(jnp.bfloat16)
            h = jax.lax.dot_general(
                xb, w1c[...], (((1,), (1,)), ((), ())),
                preferred_element_type=jnp.float32,
            )
            a = (0.5 * h * (1.0 + jax.lax.erf(h * 0.7071067811865476))).astype(
                jnp.bfloat16
            )
            yt = jax.lax.dot_general(
                a, w2c[...], (((1,), (0,)), ((), ())),
                preferred_element_type=jnp.float32,
            )
            y_ref[pl.ds(ob, BT), :] += yt
            return c2

        jax.lax.fori_loop(0, ntile, tile_body, 0)
        return carry

    jax.lax.fori_loop(0, k_total, chunk_body, 0)
    y_ref[...] = y_ref[...] * tw_ref[:, :1] + b_ref[...]


def _fused_mlp(meta, x_sorted, tw_sorted, w1r, w2r, bias2d):
    cap, d = x_sorted.shape
    e, f, _ = w1r.shape
    fch = min(FCH, f)
    return pl.pallas_call(
        _fused_body,
        grid_spec=pltpu.PrefetchScalarGridSpec(
            num_scalar_prefetch=1,
            grid=(1,),
            in_specs=[
                pl.BlockSpec((cap, d), lambda i, s: (0, 0)),
                pl.BlockSpec((cap, tw_sorted.shape[1]), lambda i, s: (0, 0)),
                pl.BlockSpec((1, d), lambda i, s: (0, 0)),
                pl.BlockSpec(memory_space=pl.ANY),
                pl.BlockSpec(memory_space=pl.ANY),
            ],
            out_specs=pl.BlockSpec((cap, d), lambda i, s: (0, 0)),
            scratch_shapes=[
                pltpu.VMEM((2, fch, d), jnp.float32),
                pltpu.VMEM((2, fch, d), jnp.float32),
                pltpu.VMEM((fch, d), jnp.bfloat16),
                pltpu.VMEM((fch, d), jnp.bfloat16),
                pltpu.SemaphoreType.DMA((2, 2)),
            ],
        ),
        out_shape=jax.ShapeDtypeStruct((cap, d), jnp.float32),
    )(meta, x_sorted, tw_sorted, bias2d, w1r, w2r)


def _routing(top_experts, t, e, cap):
    eidx = top_experts[:, 0].astype(jnp.int32)
    sidx = jnp.argsort(eidx).astype(jnp.int32)
    sorted_e = eidx[sidx]
    offsets = jnp.searchsorted(sorted_e, jnp.arange(e + 1, dtype=jnp.int32)).astype(
        jnp.int32
    )
    counts = offsets[1:] - offsets[:-1]
    aligned = ((counts + 7) // 8) * 8
    aoff = jnp.concatenate(
        [jnp.zeros((1,), jnp.int32), jnp.cumsum(aligned).astype(jnp.int32)]
    )
    pos_sorted = aoff[sorted_e] + (
        jnp.arange(t, dtype=jnp.int32) - offsets[sorted_e]
    )
    sidx_full = jnp.zeros((cap,), jnp.int32).at[pos_sorted].set(sidx)
    pos_token = jnp.zeros((t,), jnp.int32).at[sidx].set(pos_sorted)
    meta = jnp.stack([aoff[:e] // 8, counts])
    return sidx_full, pos_token, meta


def kernel(x, weights, top_weights, top_experts, w1, w2, bias):
    t, d = x.shape
    e = weights.shape[1]
    f = w1.shape[0] // e
    cap = ((t + e * 8 + BT + BT - 1) // BT) * BT

    sidx_full, pos_token, meta = _routing(top_experts, t, e, cap)

    tw128 = jnp.broadcast_to(top_weights[:, :1], (t, 128))
    x_sorted = _sc_gather_rows(x, sidx_full)
    tw_sorted = _sc_gather_tw(tw128, sidx_full)

    w1r = w1.reshape(e, f, d)
    w2r = w2.reshape(e, f, d)
    y_sorted = _fused_mlp(meta, x_sorted, tw_sorted, w1r, w2r, bias.reshape(1, d))

    return _sc_gather_rows(y_sorted, pos_token)
